# baseline (speedup 1.0000x reference)
import jax
import jax.numpy as jnp
from jax.experimental import pallas as pl


def _center_kernel(x_ref, c1_ref, c2_ref):
    x = x_ref[0]
    q = x.shape[0]
    s = jnp.sum(x, axis=0, keepdims=True)
    c = s * (1.0 / q)
    n = jnp.sum(x * x, axis=1, keepdims=True)
    xc = jnp.dot(x, c.T, preferred_element_type=jnp.float32)
    score = n - 2.0 * xc
    idx = jnp.argmax(score[:, 0], axis=0)
    c1_ref[0] = c
    c2_ref[0] = x_ref[0, pl.ds(idx, 1), :]


def kernel(hs_pair):
    L, N, Q, C = hs_pair.shape
    flat = hs_pair.reshape(L * N, Q, C)
    c1, c2 = pl.pallas_call(
        _center_kernel,
        grid=(L * N,),
        in_specs=[pl.BlockSpec((1, Q, C), lambda i: (i, 0, 0))],
        out_specs=[
            pl.BlockSpec((1, 1, C), lambda i: (i, 0, 0)),
            pl.BlockSpec((1, 1, C), lambda i: (i, 0, 0)),
        ],
        out_shape=[
            jax.ShapeDtypeStruct((L * N, 1, C), hs_pair.dtype),
            jax.ShapeDtypeStruct((L * N, 1, C), hs_pair.dtype),
        ],
    )(flat)
    return jnp.concatenate([c1, c2], axis=1).reshape(L, N, 2, C)
